# baseline (speedup 1.0000x reference)
import functools
import math

import jax
import jax.numpy as jnp
from jax import lax
from jax.experimental import pallas as pl
from jax.experimental.pallas import tpu as pltpu
from jax.experimental.pallas import tpu_sc as plsc

D_MODEL = 768
SEQ_LEN = 2048
BATCH = 4
SCALE = math.sqrt(D_MODEL)

NUM_CORES = 2
NUM_SUBCORES = 16
NUM_WORKERS = NUM_CORES * NUM_SUBCORES
LANES = 16

B_TOTAL = BATCH * SEQ_LEN
PER_WORKER = B_TOTAL // NUM_WORKERS
POS_PER_W = SEQ_LEN // NUM_WORKERS
CHUNK = 32
N_CHUNKS = PER_WORKER // CHUNK
CHUNKS_PER_BATCH = N_CHUNKS // BATCH
N_ROWBUF = 3
HC = CHUNK // 2


def _build_lookup():
    mesh = plsc.VectorSubcoreMesh(core_axis_name="c", subcore_axis_name="s")

    @functools.partial(
        pl.kernel,
        out_type=jax.ShapeDtypeStruct((B_TOTAL, D_MODEL), jnp.float32),
        mesh=mesh,
        scratch_types=[
            pltpu.VMEM((PER_WORKER,), jnp.int32),
            pltpu.VMEM((POS_PER_W, D_MODEL), jnp.float32),
            [pltpu.VMEM((CHUNK, D_MODEL), jnp.float32) for _ in range(N_ROWBUF)],
            pltpu.SemaphoreType.DMA,
            pltpu.SemaphoreType.DMA,
            [pltpu.SemaphoreType.DMA for _ in range(2 * N_ROWBUF)],
            [pltpu.SemaphoreType.DMA for _ in range(N_ROWBUF)],
        ],
    )
    def emb(ids_hbm, table_hbm, pos_hbm, out_hbm, idx_v, pos_v, rows, isem,
            psem, gsem, wsem):
        wid = lax.axis_index("s") * NUM_CORES + lax.axis_index("c")
        pos_base = pl.multiple_of(wid * POS_PER_W, POS_PER_W)

        def issue_idx(bt):
            src0 = pl.multiple_of(bt * SEQ_LEN + wid * POS_PER_W, POS_PER_W)
            return pltpu.async_copy(
                ids_hbm.at[pl.ds(src0, POS_PER_W)],
                idx_v.at[pl.ds(bt * POS_PER_W, POS_PER_W)], isem)

        ih = {0: issue_idx(0)}
        ih[0].wait()
        idx_ready = {0}

        def need_idx(c):
            bt = c // CHUNKS_PER_BATCH
            if bt not in idx_ready:
                ih[bt].wait()
                idx_ready.add(bt)

        def issue_gather(c):
            b = c % N_ROWBUF
            h0 = pltpu.async_copy(
                table_hbm.at[idx_v.at[pl.ds(c * CHUNK, HC)]],
                rows[b].at[pl.ds(0, HC)], gsem[2 * b])
            h1 = pltpu.async_copy(
                table_hbm.at[idx_v.at[pl.ds(c * CHUNK + HC, HC)]],
                rows[b].at[pl.ds(HC, HC)], gsem[2 * b + 1])
            return (h0, h1)

        def out_row0(c):
            bt, h = divmod(c, CHUNKS_PER_BATCH)
            return pl.multiple_of(
                bt * SEQ_LEN + wid * POS_PER_W + h * CHUNK, CHUNK)

        gh = {0: issue_gather(0), 1: issue_gather(1)}
        ph = pltpu.async_copy(pos_hbm.at[pl.ds(pos_base, POS_PER_W)], pos_v,
                              psem)
        for bt in range(1, BATCH):
            ih[bt] = issue_idx(bt)
        wh = {}
        pos_waited = False
        for c in range(N_CHUNKS):
            b = c % N_ROWBUF
            h = c % CHUNKS_PER_BATCH
            gh[c][0].wait()
            gh[c][1].wait()
            if not pos_waited:
                ph.wait()
                pos_waited = True

            @pl.loop(0, CHUNK)
            def _(r):
                for j in range(0, D_MODEL, LANES):
                    rows[b].at[pl.ds(r, 1), pl.ds(j, LANES)][...] = (
                        rows[b].at[pl.ds(r, 1), pl.ds(j, LANES)][...] * SCALE
                        + pos_v.at[pl.ds(h * CHUNK + r, 1),
                                   pl.ds(j, LANES)][...]
                    )

            wh[c] = pltpu.async_copy(rows[b], out_hbm.at[pl.ds(out_row0(c),
                                                               CHUNK)],
                                     wsem[b])
            if c + 2 < N_CHUNKS:
                if c >= 1:
                    wh[c - 1].wait()
                need_idx(c + 2)
                gh[c + 2] = issue_gather(c + 2)

        for c in range(N_CHUNKS - N_ROWBUF, N_CHUNKS):
            wh[c].wait()

    return emb


_lookup = _build_lookup()


@jax.jit
def kernel(input_ids, token_table, pos_table):
    batch, seq_len = input_ids.shape
    flat_ids = input_ids.reshape(-1).astype(jnp.int32)
    out = _lookup(flat_ids, token_table, pos_table)
    return out.reshape(batch, seq_len, D_MODEL)


# speedup vs baseline: 1.1521x; 1.1521x over previous
import functools
import math

import jax
import jax.numpy as jnp
from jax import lax
from jax.experimental import pallas as pl
from jax.experimental.pallas import tpu as pltpu
from jax.experimental.pallas import tpu_sc as plsc

D_MODEL = 768
SEQ_LEN = 2048
SCALE = math.sqrt(D_MODEL)

NUM_CORES = 2
NUM_SUBCORES = 16
NUM_WORKERS = NUM_CORES * NUM_SUBCORES
LANES = 16

B_TOTAL = 4 * SEQ_LEN
PER_WORKER = B_TOTAL // NUM_WORKERS
CHUNK = 32
N_CHUNKS = PER_WORKER // CHUNK
N_ROWBUF = 3
N_POSBUF = 2


def _build_lookup():
    mesh = plsc.VectorSubcoreMesh(core_axis_name="c", subcore_axis_name="s")

    @functools.partial(
        pl.kernel,
        out_type=jax.ShapeDtypeStruct((B_TOTAL, D_MODEL), jnp.float32),
        mesh=mesh,
        scratch_types=[
            pltpu.VMEM((PER_WORKER,), jnp.int32),
            [pltpu.VMEM((CHUNK, D_MODEL), jnp.float32) for _ in range(N_ROWBUF)],
            [pltpu.VMEM((CHUNK, D_MODEL), jnp.float32) for _ in range(N_POSBUF)],
            [pltpu.SemaphoreType.DMA for _ in range(N_ROWBUF)],
            [pltpu.SemaphoreType.DMA for _ in range(N_POSBUF)],
            [pltpu.SemaphoreType.DMA for _ in range(N_ROWBUF)],
        ],
    )
    def emb(ids_hbm, table_hbm, pos_hbm, out_hbm, idx_v, rows, posb, gsem,
            psem, wsem):
        wid = lax.axis_index("s") * NUM_CORES + lax.axis_index("c")
        base = pl.multiple_of(wid * PER_WORKER, PER_WORKER)
        pltpu.sync_copy(ids_hbm.at[pl.ds(base, PER_WORKER)], idx_v)

        def issue_gather(c):
            b = c % N_ROWBUF
            return pltpu.async_copy(
                table_hbm.at[idx_v.at[pl.ds(c * CHUNK, CHUNK)]], rows[b],
                gsem[b])

        def issue_pos(c):
            b = c % N_POSBUF
            pos0 = pl.multiple_of(
                lax.rem(base + c * CHUNK, SEQ_LEN), CHUNK)
            return pltpu.async_copy(
                pos_hbm.at[pl.ds(pos0, CHUNK)], posb[b], psem[b])

        gh = {0: issue_gather(0), 1: issue_gather(1)}
        ph = {0: issue_pos(0), 1: issue_pos(1)}
        wh = {}
        for c in range(N_CHUNKS):
            b = c % N_ROWBUF
            pb = c % N_POSBUF
            gh[c].wait()
            ph[c].wait()

            @pl.loop(0, CHUNK)
            def _(r):
                for j in range(0, D_MODEL, LANES):
                    rows[b].at[pl.ds(r, 1), pl.ds(j, LANES)][...] = (
                        rows[b].at[pl.ds(r, 1), pl.ds(j, LANES)][...] * SCALE
                        + posb[pb].at[pl.ds(r, 1), pl.ds(j, LANES)][...]
                    )

            row0 = pl.multiple_of(base + c * CHUNK, CHUNK)
            wh[c] = pltpu.async_copy(rows[b], out_hbm.at[pl.ds(row0, CHUNK)],
                                     wsem[b])
            if c + 2 < N_CHUNKS:
                if c >= 1:
                    wh[c - 1].wait()
                gh[c + 2] = issue_gather(c + 2)
                ph[c + 2] = issue_pos(c + 2)

        for c in range(N_CHUNKS - N_ROWBUF, N_CHUNKS):
            wh[c].wait()

    return emb


_lookup = _build_lookup()


@jax.jit
def kernel(input_ids, token_table, pos_table):
    batch, seq_len = input_ids.shape
    flat_ids = input_ids.reshape(-1).astype(jnp.int32)
    out = _lookup(flat_ids, token_table, pos_table)
    return out.reshape(batch, seq_len, D_MODEL)


# speedup vs baseline: 1.1574x; 1.0046x over previous
import functools
import math

import jax
import jax.numpy as jnp
from jax import lax
from jax.experimental import pallas as pl
from jax.experimental.pallas import tpu as pltpu
from jax.experimental.pallas import tpu_sc as plsc

D_MODEL = 768
SEQ_LEN = 2048
SCALE = math.sqrt(D_MODEL)

NUM_CORES = 2
NUM_SUBCORES = 16
NUM_WORKERS = NUM_CORES * NUM_SUBCORES
LANES = 16

B_TOTAL = 4 * SEQ_LEN
PER_WORKER = B_TOTAL // NUM_WORKERS
CHUNK = 32
N_CHUNKS = PER_WORKER // CHUNK
N_ROWBUF = 3
N_POSBUF = 2


def _build_lookup():
    mesh = plsc.VectorSubcoreMesh(core_axis_name="c", subcore_axis_name="s")

    @functools.partial(
        pl.kernel,
        out_type=jax.ShapeDtypeStruct((B_TOTAL, D_MODEL), jnp.float32),
        mesh=mesh,
        scratch_types=[
            pltpu.VMEM((PER_WORKER,), jnp.int32),
            [pltpu.VMEM((CHUNK, D_MODEL), jnp.float32) for _ in range(N_ROWBUF)],
            [pltpu.VMEM((CHUNK, D_MODEL), jnp.float32) for _ in range(N_POSBUF)],
            [pltpu.SemaphoreType.DMA for _ in range(2 * N_ROWBUF)],
            [pltpu.SemaphoreType.DMA for _ in range(N_POSBUF)],
            [pltpu.SemaphoreType.DMA for _ in range(N_ROWBUF)],
        ],
    )
    def emb(ids_hbm, table_hbm, pos_hbm, out_hbm, idx_v, rows, posb, gsem,
            psem, wsem):
        wid = lax.axis_index("s") * NUM_CORES + lax.axis_index("c")
        base = pl.multiple_of(wid * PER_WORKER, PER_WORKER)
        pltpu.sync_copy(ids_hbm.at[pl.ds(base, PER_WORKER)], idx_v)

        HC = CHUNK // 2

        def issue_gather(c):
            b = c % N_ROWBUF
            h0 = pltpu.async_copy(
                table_hbm.at[idx_v.at[pl.ds(c * CHUNK, HC)]],
                rows[b].at[pl.ds(0, HC)], gsem[2 * b])
            h1 = pltpu.async_copy(
                table_hbm.at[idx_v.at[pl.ds(c * CHUNK + HC, HC)]],
                rows[b].at[pl.ds(HC, HC)], gsem[2 * b + 1])
            return (h0, h1)

        def issue_pos(c):
            b = c % N_POSBUF
            pos0 = pl.multiple_of(
                lax.rem(base + c * CHUNK, SEQ_LEN), CHUNK)
            return pltpu.async_copy(
                pos_hbm.at[pl.ds(pos0, CHUNK)], posb[b], psem[b])

        gh = {0: issue_gather(0), 1: issue_gather(1)}
        ph = {0: issue_pos(0), 1: issue_pos(1)}
        wh = {}
        for c in range(N_CHUNKS):
            b = c % N_ROWBUF
            pb = c % N_POSBUF
            gh[c][0].wait()
            gh[c][1].wait()
            ph[c].wait()

            @pl.loop(0, CHUNK)
            def _(r):
                for j in range(0, D_MODEL, LANES):
                    rows[b].at[pl.ds(r, 1), pl.ds(j, LANES)][...] = (
                        rows[b].at[pl.ds(r, 1), pl.ds(j, LANES)][...] * SCALE
                        + posb[pb].at[pl.ds(r, 1), pl.ds(j, LANES)][...]
                    )

            row0 = pl.multiple_of(base + c * CHUNK, CHUNK)
            wh[c] = pltpu.async_copy(rows[b], out_hbm.at[pl.ds(row0, CHUNK)],
                                     wsem[b])
            if c + 2 < N_CHUNKS:
                if c >= 1:
                    wh[c - 1].wait()
                gh[c + 2] = issue_gather(c + 2)
                ph[c + 2] = issue_pos(c + 2)

        for c in range(N_CHUNKS - N_ROWBUF, N_CHUNKS):
            wh[c].wait()

    return emb


_lookup = _build_lookup()


@jax.jit
def kernel(input_ids, token_table, pos_table):
    batch, seq_len = input_ids.shape
    flat_ids = input_ids.reshape(-1).astype(jnp.int32)
    out = _lookup(flat_ids, token_table, pos_table)
    return out.reshape(batch, seq_len, D_MODEL)
